# baseline (speedup 1.0000x reference)
import functools

import jax
import jax.numpy as jnp
from jax import lax
from jax.experimental import pallas as pl
from jax.experimental.pallas import tpu as pltpu
from jax.experimental.pallas import tpu_sc as plsc

B, T, N, E = 32, 64, 80, 2560
OUT, HID = 8, 24
G = B * T
NC, NS, L = 2, 16, 16
NW = NC * NS
GPW = G // NW
NCHUNK = E // L
NXC = N // L


def _gat_sc(xf, wf, src, dst, params):
    mesh = plsc.VectorSubcoreMesh(core_axis_name="c", subcore_axis_name="s")

    @functools.partial(
        pl.kernel, mesh=mesh,
        out_type=jax.ShapeDtypeStruct((G,), jnp.float32),
        scratch_types=[
            pltpu.VMEM((E,), jnp.int32),
            pltpu.VMEM((E,), jnp.int32),
            pltpu.VMEM((GPW * N,), jnp.float32),
            pltpu.VMEM((2, E), jnp.float32),
            pltpu.VMEM((N,), jnp.float32),
            pltpu.VMEM((N,), jnp.float32),
            pltpu.VMEM((GPW,), jnp.float32),
            pltpu.VMEM((L,), jnp.float32),
            pltpu.SemaphoreType.DMA,
            pltpu.SemaphoreType.DMA,
        ],
        compiler_params=pltpu.CompilerParams(needs_layout_passes=False,
                                             use_tc_tiling_on_sc=True),
    )
    def k(x_hbm, w_hbm, src_hbm, dst_hbm, par_hbm, m_hbm,
          srcv, dstv, xblk, wbuf, denom, num, mout, parv, sem0, sem1):
        wid = lax.axis_index("s") * NC + lax.axis_index("c")
        base = wid * GPW
        pltpu.sync_copy(src_hbm, srcv)
        pltpu.sync_copy(dst_hbm, dstv)
        pltpu.sync_copy(par_hbm, parv)
        pltpu.sync_copy(x_hbm.at[pl.ds(base * N, GPW * N)], xblk)
        zero16 = jnp.zeros((L,), jnp.float32)
        izero = lax.iota(jnp.int32, L) * 0
        dn = lax.GatherDimensionNumbers(offset_dims=(), collapsed_slice_dims=(0,),
                                        start_index_map=(0,))

        def bcast(v, j):
            return lax.gather(v, (izero + j)[:, None], dn, slice_sizes=(1,),
                              mode=lax.GatherScatterMode.PROMISE_IN_BOUNDS)

        par = parv[pl.ds(0, L)]
        cl = bcast(par, 0)
        cr = bcast(par, 1)
        ce = bcast(par, 2)
        kcoef = bcast(par, 3)
        kbias = bcast(par, 4)

        def process_graph(gi, wb):
            for cj in range(NXC):
                denom[pl.ds(cj * L, L)] = zero16
                num[pl.ds(cj * L, L)] = zero16
            gbase = gi * N
            amax = jnp.abs(xblk[pl.ds(gbase, L)])
            for cj in range(1, NXC):
                amax = jnp.maximum(amax, jnp.abs(xblk[pl.ds(gbase + cj * L, L)]))
            for sh in (8, 4, 2, 1):
                idx = lax.iota(jnp.int32, L) ^ sh
                amax = jnp.maximum(amax, lax.gather(
                    amax, idx[:, None], dn, slice_sizes=(1,),
                    mode=lax.GatherScatterMode.PROMISE_IN_BOUNDS))
            K = kcoef * amax + kbias

            @plsc.parallel_loop(0, NCHUNK, 1, unroll=8)
            def chunk_body(ci):
                sl = pl.ds(ci * L, L)
                si = srcv[sl]
                di = dstv[sl]
                xs = plsc.load_gather(xblk, [si + gbase])
                xd = plsc.load_gather(xblk, [di + gbase])
                wv = wbuf[wb, sl]
                e = cl * xs + cr * xd + ce * wv
                e = jnp.maximum(e, 0.2 * e)
                p = jnp.exp(e - K)
                plsc.addupdate_scatter(denom, [di], p)
                plsc.addupdate_scatter(num, [di], p * xs)

            s = zero16
            for cj in range(NXC):
                sl = pl.ds(cj * L, L)
                s = s + num[sl] / (denom[sl] + 1e-9)
            for sh in (8, 4, 2, 1):
                idx = lax.iota(jnp.int32, L) ^ sh
                s = s + lax.gather(s, idx[:, None], dn, slice_sizes=(1,),
                                   mode=lax.GatherScatterMode.PROMISE_IN_BOUNDS)
            plsc.store_scatter(mout, [izero + gi], s * (1.0 / N))

        def wcopy(g, b, sem):
            return pltpu.make_async_copy(w_hbm.at[pl.ds(g, 1)],
                                         wbuf.at[pl.ds(b, 1)], sem)

        wcopy(base, 0, sem0).start()

        def pair_body(gp, _):
            g0 = 2 * gp
            wcopy(base + g0, 0, sem0).wait()
            wcopy(base + g0 + 1, 1, sem1).start()
            process_graph(g0, 0)
            wcopy(base + g0 + 1, 1, sem1).wait()

            @pl.when(gp + 1 < GPW // 2)
            def _prefetch():
                wcopy(base + g0 + 2, 0, sem0).start()

            process_graph(g0 + 1, 1)
            return _

        lax.fori_loop(0, GPW // 2, pair_body, 0)
        pltpu.sync_copy(mout, m_hbm.at[pl.ds(base, GPW)])

    return k(xf, wf, src, dst, params)


def _lstm_tc(m_tb1, vins, vbs, whhs, M2T, b2):
    def body(m_ref, vi_ref, vf_ref, vg_ref, vo_ref,
             bi_ref, bf_ref, bg_ref, bo_ref,
             wi_ref, wf_ref, wg_ref, wo_ref, m2_ref, b2_ref, out_ref):
        vi, vf, vg, vo = vi_ref[...], vf_ref[...], vg_ref[...], vo_ref[...]
        bi, bf, bg, bo = bi_ref[...], bf_ref[...], bg_ref[...], bo_ref[...]
        wi, wf, wg, wo = wi_ref[...], wf_ref[...], wg_ref[...], wo_ref[...]
        m2 = m2_ref[...]
        b2v = b2_ref[...]

        def dot(a, b):
            return lax.dot_general(a, b, (((1,), (0,)), ((), ())),
                                   preferred_element_type=jnp.float32)

        def step(t, carry):
            h, c = carry
            mt = m_ref[t]
            i = jax.nn.sigmoid(mt * vi + bi + dot(h, wi))
            f = jax.nn.sigmoid(mt * vf + bf + dot(h, wf))
            g = jnp.tanh(mt * vg + bg + dot(h, wg))
            o = jax.nn.sigmoid(mt * vo + bo + dot(h, wo))
            c = f * c + i * g
            h = o * jnp.tanh(c)
            out_ref[:, t, :] = dot(h, m2) + b2v
            return (h, c)

        h0 = jnp.zeros((B, HID), jnp.float32)
        c0 = jnp.zeros((B, HID), jnp.float32)
        lax.fori_loop(0, T, step, (h0, c0))

    return pl.pallas_call(
        body,
        out_shape=jax.ShapeDtypeStruct((B, T, 20), jnp.float32),
    )(m_tb1, *vins, *vbs, *whhs, M2T, b2)


def kernel(x, edge_index, edge_weight, W_node, a_l, a_r, W_edge, a_e,
           gat_bias, W_ih, W_hh, b_ih, b_hh, fc_W, fc_b, fcc_W, fcc_b):
    xf = x.reshape(G * N)
    wf = edge_weight.reshape(G, E)
    src = edge_index[0]
    dst = edge_index[1]

    cl = (W_node[0] * a_l).sum()
    cr = (W_node[0] * a_r).sum()
    ce = (W_edge[0] * a_e).sum()
    z = cl * 0
    params = jnp.stack([cl, cr, ce,
                        jnp.abs(cl) + jnp.abs(cr), jnp.abs(ce),
                        z, z, z, z, z, z, z, z, z, z, z]).astype(jnp.float32)

    m = _gat_sc(xf, wf, src, dst, params)

    v_in = (W_node[0] @ W_ih.T).reshape(1, 4 * HID)
    v_b = (gat_bias @ W_ih.T + b_ih + b_hh).reshape(1, 4 * HID)
    M2T = fc_W.T @ fcc_W.T
    b2 = (fc_b @ fcc_W.T + fcc_b).reshape(1, 20)
    vins = [v_in[:, j * HID:(j + 1) * HID] for j in range(4)]
    vbs = [v_b[:, j * HID:(j + 1) * HID] for j in range(4)]
    whhT = W_hh.T
    whhs = [whhT[:, j * HID:(j + 1) * HID] for j in range(4)]

    m_tb1 = m.reshape(B, T).T.reshape(T, B, 1)
    out = _lstm_tc(m_tb1, vins, vbs, whhs, M2T, b2)
    return out.reshape(G, 20)


# speedup vs baseline: 1.0954x; 1.0954x over previous
import functools

import jax
import jax.numpy as jnp
from jax import lax
from jax.experimental import pallas as pl
from jax.experimental.pallas import tpu as pltpu
from jax.experimental.pallas import tpu_sc as plsc

B, T, N, E = 32, 64, 80, 2560
OUT, HID = 8, 24
G = B * T
NC, NS, L = 2, 16, 16
NW = NC * NS
GPW = G // NW
NCHUNK = E // L
NXC = N // L


def _gat_sc(xf, wf, src, dst, params):
    mesh = plsc.VectorSubcoreMesh(core_axis_name="c", subcore_axis_name="s")

    @functools.partial(
        pl.kernel, mesh=mesh,
        out_type=jax.ShapeDtypeStruct((G,), jnp.float32),
        scratch_types=[
            pltpu.VMEM((E,), jnp.int32),
            pltpu.VMEM((E,), jnp.int32),
            pltpu.VMEM((GPW * N,), jnp.float32),
            pltpu.VMEM((2 * E,), jnp.float32),
            pltpu.VMEM((N,), jnp.float32),
            pltpu.VMEM((N,), jnp.float32),
            pltpu.VMEM((GPW,), jnp.float32),
            pltpu.VMEM((L,), jnp.float32),
            pltpu.SemaphoreType.DMA,
            pltpu.SemaphoreType.DMA,
        ],
        compiler_params=pltpu.CompilerParams(needs_layout_passes=False),
    )
    def k(x_hbm, w_hbm, src_hbm, dst_hbm, par_hbm, m_hbm,
          srcv, dstv, xblk, wbuf, denom, num, mout, parv, sem0, sem1):
        wid = lax.axis_index("s") * NC + lax.axis_index("c")
        base = wid * GPW
        pltpu.sync_copy(src_hbm, srcv)
        pltpu.sync_copy(dst_hbm, dstv)
        pltpu.sync_copy(par_hbm, parv)
        pltpu.sync_copy(x_hbm.at[pl.ds(base * N, GPW * N)], xblk)
        zero16 = jnp.zeros((L,), jnp.float32)
        izero = lax.iota(jnp.int32, L) * 0
        dn = lax.GatherDimensionNumbers(offset_dims=(), collapsed_slice_dims=(0,),
                                        start_index_map=(0,))

        def bcast(v, j):
            return lax.gather(v, (izero + j)[:, None], dn, slice_sizes=(1,),
                              mode=lax.GatherScatterMode.PROMISE_IN_BOUNDS)

        par = parv[pl.ds(0, L)]
        cl = bcast(par, 0)
        cr = bcast(par, 1)
        ce = bcast(par, 2)
        kcoef = bcast(par, 3)
        kbias = bcast(par, 4)

        def process_graph(gi, wb):
            for cj in range(NXC):
                denom[pl.ds(cj * L, L)] = zero16
                num[pl.ds(cj * L, L)] = zero16
            gbase = gi * N
            amax = jnp.abs(xblk[pl.ds(gbase, L)])
            for cj in range(1, NXC):
                amax = jnp.maximum(amax, jnp.abs(xblk[pl.ds(gbase + cj * L, L)]))
            for sh in (8, 4, 2, 1):
                idx = lax.iota(jnp.int32, L) ^ sh
                amax = jnp.maximum(amax, lax.gather(
                    amax, idx[:, None], dn, slice_sizes=(1,),
                    mode=lax.GatherScatterMode.PROMISE_IN_BOUNDS))
            K = kcoef * amax + kbias

            @plsc.parallel_loop(0, NCHUNK, 1, unroll=8)
            def chunk_body(ci):
                sl = pl.ds(ci * L, L)
                si = srcv[sl]
                di = dstv[sl]
                xs = plsc.load_gather(xblk, [si + gbase])
                xd = plsc.load_gather(xblk, [di + gbase])
                wv = wbuf[pl.ds(wb * E + ci * L, L)]
                e = cl * xs + cr * xd + ce * wv
                e = jnp.maximum(e, 0.2 * e)
                p = jnp.exp(e - K)
                plsc.addupdate_scatter(denom, [di], p)
                plsc.addupdate_scatter(num, [di], p * xs)

            s = zero16
            for cj in range(NXC):
                sl = pl.ds(cj * L, L)
                s = s + num[sl] / (denom[sl] + 1e-9)
            for sh in (8, 4, 2, 1):
                idx = lax.iota(jnp.int32, L) ^ sh
                s = s + lax.gather(s, idx[:, None], dn, slice_sizes=(1,),
                                   mode=lax.GatherScatterMode.PROMISE_IN_BOUNDS)
            plsc.store_scatter(mout, [izero + gi], s * (1.0 / N))

        def wcopy(g, b, sem):
            return pltpu.make_async_copy(w_hbm.at[pl.ds(g * E, E)],
                                         wbuf.at[pl.ds(b * E, E)], sem)

        wcopy(base, 0, sem0).start()

        def pair_body(gp, _):
            g0 = 2 * gp
            wcopy(base + g0, 0, sem0).wait()
            wcopy(base + g0 + 1, 1, sem1).start()
            process_graph(g0, 0)
            wcopy(base + g0 + 1, 1, sem1).wait()

            @pl.when(gp + 1 < GPW // 2)
            def _prefetch():
                wcopy(base + g0 + 2, 0, sem0).start()

            process_graph(g0 + 1, 1)
            return _

        lax.fori_loop(0, GPW // 2, pair_body, 0)
        pltpu.sync_copy(mout, m_hbm.at[pl.ds(base, GPW)])

    return k(xf, wf, src, dst, params)


def _lstm_tc(m_tb1, vins, vbs, whhs, M2T, b2):
    def body(m_ref, vi_ref, vf_ref, vg_ref, vo_ref,
             bi_ref, bf_ref, bg_ref, bo_ref,
             wi_ref, wf_ref, wg_ref, wo_ref, m2_ref, b2_ref, out_ref):
        vi, vf, vg, vo = vi_ref[...], vf_ref[...], vg_ref[...], vo_ref[...]
        bi, bf, bg, bo = bi_ref[...], bf_ref[...], bg_ref[...], bo_ref[...]
        wi, wf, wg, wo = wi_ref[...], wf_ref[...], wg_ref[...], wo_ref[...]
        m2 = m2_ref[...]
        b2v = b2_ref[...]

        def dot(a, b):
            return lax.dot_general(a, b, (((1,), (0,)), ((), ())),
                                   preferred_element_type=jnp.float32)

        def step(t, carry):
            h, c = carry
            mt = m_ref[t]
            i = jax.nn.sigmoid(mt * vi + bi + dot(h, wi))
            f = jax.nn.sigmoid(mt * vf + bf + dot(h, wf))
            g = jnp.tanh(mt * vg + bg + dot(h, wg))
            o = jax.nn.sigmoid(mt * vo + bo + dot(h, wo))
            c = f * c + i * g
            h = o * jnp.tanh(c)
            out_ref[:, t, :] = dot(h, m2) + b2v
            return (h, c)

        h0 = jnp.zeros((B, HID), jnp.float32)
        c0 = jnp.zeros((B, HID), jnp.float32)
        lax.fori_loop(0, T, step, (h0, c0))

    return pl.pallas_call(
        body,
        out_shape=jax.ShapeDtypeStruct((B, T, 20), jnp.float32),
    )(m_tb1, *vins, *vbs, *whhs, M2T, b2)


def kernel(x, edge_index, edge_weight, W_node, a_l, a_r, W_edge, a_e,
           gat_bias, W_ih, W_hh, b_ih, b_hh, fc_W, fc_b, fcc_W, fcc_b):
    xf = x.reshape(G * N)
    wf = edge_weight.reshape(G * E)
    src = edge_index[0]
    dst = edge_index[1]

    cl = (W_node[0] * a_l).sum()
    cr = (W_node[0] * a_r).sum()
    ce = (W_edge[0] * a_e).sum()
    z = cl * 0
    params = jnp.stack([cl, cr, ce,
                        jnp.abs(cl) + jnp.abs(cr), jnp.abs(ce),
                        z, z, z, z, z, z, z, z, z, z, z]).astype(jnp.float32)

    m = _gat_sc(xf, wf, src, dst, params)

    v_in = (W_node[0] @ W_ih.T).reshape(1, 4 * HID)
    v_b = (gat_bias @ W_ih.T + b_ih + b_hh).reshape(1, 4 * HID)
    M2T = fc_W.T @ fcc_W.T
    b2 = (fc_b @ fcc_W.T + fcc_b).reshape(1, 20)
    vins = [v_in[:, j * HID:(j + 1) * HID] for j in range(4)]
    vbs = [v_b[:, j * HID:(j + 1) * HID] for j in range(4)]
    whhT = W_hh.T
    whhs = [whhT[:, j * HID:(j + 1) * HID] for j in range(4)]

    m_tb1 = m.reshape(B, T).T.reshape(T, B, 1)
    out = _lstm_tc(m_tb1, vins, vbs, whhs, M2T, b2)
    return out.reshape(G, 20)
